# Initial kernel scaffold; baseline (speedup 1.0000x reference)
import jax
import jax.numpy as jnp
from jax.experimental import pallas as pl


def kernel(feat1, feat2, msk, samp_bias1, samp_bias2, edge_index, W_prep, W_node, Wa, va, W_fc, b_fc, W_d):
    raise NotImplementedError("write your pallas kernel here")


# speedup vs baseline: 7.2768x; 7.2768x over previous
import functools

import jax
import jax.numpy as jnp
from jax import lax
from jax.experimental import pallas as pl
from jax.experimental.pallas import tpu as pltpu
from jax.experimental.pallas import tpu_sc as plsc

N = 10000
E = 320000
D = 128
N_MP = 2
N_CLASSES = 16
ATTN_DIM = 64

NC = 2
NT = 16
EPT = E // NT
K = 80
NCH = EPT // K
NB = 10
CPB = NCH // NB
NP = 10240
RPT = NP // NT
ZAR = 64
ZDR = 128
DEGW = 16


def _prep_body(f1, f2, w, out):
    wv = w[...]
    out[0] = jnp.maximum(jnp.dot(f1[...], wv, preferred_element_type=jnp.float32), 0.0)
    out[1] = jnp.maximum(jnp.dot(f2[...], wv, preferred_element_type=jnp.float32), 0.0)


_prep = pl.pallas_call(
    _prep_body,
    out_shape=jax.ShapeDtypeStruct((2, N, D), jnp.float32),
)


_mesh = plsc.VectorSubcoreMesh(core_axis_name="c", subcore_axis_name="s")


@functools.partial(
    pl.kernel,
    out_type=(
        jax.ShapeDtypeStruct((NC, NP, D), jnp.float32),
        jax.ShapeDtypeStruct((NC, NP, DEGW), jnp.float32),
    ),
    mesh=_mesh,
    compiler_params=pltpu.CompilerParams(use_tc_tiling_on_sc=False),
    scratch_types=(
        pltpu.VMEM((CPB, K), jnp.int32),
        pltpu.VMEM((CPB, K), jnp.int32),
        pltpu.VMEM((K, D), jnp.float32),
        pltpu.VMEM((K, DEGW), jnp.float32),
        pltpu.VMEM((ZAR, D), jnp.float32),
        pltpu.VMEM((ZDR, DEGW), jnp.float32),
        pltpu.VMEM_SHARED((NP, D), jnp.float32),
        pltpu.VMEM_SHARED((NP, DEGW), jnp.float32),
        pltpu.SemaphoreType.DMA,
    ),
)
def _sc_aggregate(xcat, srcs, dsts, ones_h, zagg_h, zdeg_h, agg_out, deg_out,
                  src_v, dst_v, rows_v, ones_v, zagg_v, zdeg_v,
                  agg_sh, deg_sh, sem):
    c = lax.axis_index("c")
    s = lax.axis_index("s")

    pltpu.sync_copy(ones_h, ones_v)
    pltpu.sync_copy(zagg_h, zagg_v)
    pltpu.sync_copy(zdeg_h, zdeg_v)
    for j in range(RPT // ZAR):
        pltpu.sync_copy(zagg_v, agg_sh.at[pl.ds(s * RPT + j * ZAR, ZAR)])
    for j in range(RPT // ZDR):
        pltpu.sync_copy(zdeg_v, deg_sh.at[pl.ds(s * RPT + j * ZDR, ZDR)])
    plsc.subcore_barrier()

    def blk_body(blk, carry):
        pltpu.sync_copy(srcs.at[c].at[s].at[blk], src_v)
        pltpu.sync_copy(dsts.at[s].at[blk], dst_v)
        do_deg = jnp.where(c == 0, blk < NB // 2, blk >= NB // 2)

        def chunk_body(i, carry2):
            pltpu.async_copy(xcat.at[src_v.at[i]], rows_v, sem).wait()
            pltpu.sync_copy(rows_v, agg_sh.at[dst_v.at[i]], add=True)

            @pl.when(do_deg)
            def _():
                pltpu.sync_copy(ones_v, deg_sh.at[dst_v.at[i]], add=True)

            return carry2

        lax.fori_loop(0, CPB, chunk_body, ())
        return carry

    lax.fori_loop(0, NB, blk_body, ())
    plsc.subcore_barrier()

    for j in range(RPT // ZAR):
        base = s * RPT + j * ZAR
        pltpu.sync_copy(agg_sh.at[pl.ds(base, ZAR)], zagg_v)
        pltpu.sync_copy(zagg_v, agg_out.at[c].at[pl.ds(base, ZAR)])
    for j in range(RPT // ZDR):
        base = s * RPT + j * ZDR
        pltpu.sync_copy(deg_sh.at[pl.ds(base, ZDR)], zdeg_v)
        pltpu.sync_copy(zdeg_v, deg_out.at[c].at[pl.ds(base, ZDR)])


def _tail_body(aggs, degp, mskr, sb1, sb2, Wn, Wa_, va_, Wfc, bfc, Wd,
               preds_o, w_o, reg_o):
    dp = degp[...]
    deg = dp[0, :N, 0:1] + dp[1, :N, 0:1]
    inv = 1.0 / jnp.maximum(deg, 1.0)
    ag = aggs[...]
    agg1 = ag[0, :N] * inv
    agg2 = ag[1, :N] * inv
    Wnv = Wn[...]
    dot = functools.partial(jnp.dot, preferred_element_type=jnp.float32)
    h10 = jnp.maximum(dot(agg1, Wnv[0]), 0.0)
    h11 = jnp.maximum(dot(agg1, Wnv[1]), 0.0)
    h20 = jnp.maximum(dot(agg2, Wnv[0]), 0.0)
    h21 = jnp.maximum(dot(agg2, Wnv[1]), 0.0)

    Wav = Wa_[...]
    vav = va_[...]
    s0 = dot(jnp.tanh(dot(h10, Wav)), vav)
    s1 = dot(jnp.tanh(dot(h11, Wav)), vav)
    mx = jnp.maximum(s0, s1)
    e0 = jnp.exp(s0 - mx)
    e1 = jnp.exp(s1 - mx)
    z = e0 + e1
    a0 = e0 / z
    a1 = e1 / z
    outp = a0 * h10 + a1 * h11
    preds_o[...] = dot(outp, Wfc[...]) + bfc[...]
    w_o[...] = jnp.concatenate(
        [(jnp.sum(a0) / N).reshape(1, 1), (jnp.sum(a1) / N).reshape(1, 1)],
        axis=1)

    m = mskr[...]
    sm = jnp.sum(m)
    c0 = jax.nn.sigmoid(jnp.sum(h10 * m, axis=0, keepdims=True) / sm)
    c1 = jax.nn.sigmoid(jnp.sum(h11 * m, axis=0, keepdims=True) / sm)
    Wdv = Wd[...]
    cdims = (((1,), (1,)), ((), ()))
    u0 = lax.dot_general(c0, Wdv, cdims,
                         preferred_element_type=jnp.float32)
    u1 = lax.dot_general(c1, Wdv, cdims,
                         preferred_element_type=jnp.float32)
    s10 = lax.dot_general(u0, h10, cdims,
                          preferred_element_type=jnp.float32) + sb1[...]
    s11 = lax.dot_general(u1, h11, cdims,
                          preferred_element_type=jnp.float32) + sb1[...]
    s20 = lax.dot_general(u0, h20, cdims,
                          preferred_element_type=jnp.float32) + sb2[...]
    s21 = lax.dot_general(u1, h21, cdims,
                          preferred_element_type=jnp.float32) + sb2[...]
    reg_o[...] = jnp.concatenate(
        [jnp.concatenate([s10, s11], axis=0),
         jnp.concatenate([s20, s21], axis=0)], axis=1)


_tail = pl.pallas_call(
    _tail_body,
    out_shape=(
        jax.ShapeDtypeStruct((N, N_CLASSES), jnp.float32),
        jax.ShapeDtypeStruct((1, N_MP), jnp.float32),
        jax.ShapeDtypeStruct((N_MP, 2 * N), jnp.float32),
    ),
)


def kernel(feat1, feat2, msk, samp_bias1, samp_bias2, edge_index,
           W_prep, W_node, Wa, va, W_fc, b_fc, W_d):
    x2 = _prep(feat1, feat2, W_prep)
    xcat = x2.reshape(2 * N, D)
    src = edge_index[0]
    dst = edge_index[1]
    srcs = jnp.stack([src, src + N]).reshape(NC, NT, NB, CPB, K)
    dsts = dst.reshape(NT, NB, CPB, K)
    ones_h = jnp.ones((K, DEGW), jnp.float32)
    zagg_h = jnp.zeros((ZAR, D), jnp.float32)
    zdeg_h = jnp.zeros((ZDR, DEGW), jnp.float32)
    aggs, degp = _sc_aggregate(xcat, srcs, dsts, ones_h, zagg_h, zdeg_h)
    preds, w_o, reg = _tail(
        aggs, degp, msk.reshape(N, 1),
        samp_bias1.reshape(1, N), samp_bias2.reshape(1, N),
        W_node, Wa, va.reshape(ATTN_DIM, 1), W_fc,
        b_fc.reshape(1, N_CLASSES), W_d)
    return preds, w_o.reshape(N_MP), reg


# speedup vs baseline: 10.1080x; 1.3891x over previous
import functools

import jax
import jax.numpy as jnp
from jax import lax
from jax.experimental import pallas as pl
from jax.experimental.pallas import tpu as pltpu
from jax.experimental.pallas import tpu_sc as plsc

N = 10000
E = 320000
D = 128
DW = 144
N_MP = 2
N_CLASSES = 16
ATTN_DIM = 64

NC = 2
NT = 16
EPT = E // NT
K = 80
NCH = EPT // K
NB = 5
CPB = NCH // NB
NP = 10240
RPT = NP // NT
DEG_COL = 128


def _prep_body(f1, f2, w, out):
    wv = w[...]
    one = jnp.ones((N, DW - D), jnp.float32)
    r1 = jnp.maximum(jnp.dot(f1[...], wv, preferred_element_type=jnp.float32), 0.0)
    r2 = jnp.maximum(jnp.dot(f2[...], wv, preferred_element_type=jnp.float32), 0.0)
    out[0] = jnp.concatenate([r1, one], axis=1)
    out[1] = jnp.concatenate([r2, one], axis=1)


_prep = pl.pallas_call(
    _prep_body,
    out_shape=jax.ShapeDtypeStruct((2, N, DW), jnp.float32),
)


_mesh = plsc.VectorSubcoreMesh(core_axis_name="c", subcore_axis_name="s")


@functools.partial(
    pl.kernel,
    out_type=jax.ShapeDtypeStruct((NC, NP, DW), jnp.float32),
    mesh=_mesh,
    compiler_params=pltpu.CompilerParams(use_tc_tiling_on_sc=False),
    scratch_types=(
        pltpu.VMEM((CPB, K), jnp.int32),
        pltpu.VMEM((CPB, K), jnp.int32),
        pltpu.VMEM((2, K, DW), jnp.float32),
        pltpu.VMEM_SHARED((NP, DW), jnp.float32),
        pltpu.SemaphoreType.DMA,
        pltpu.SemaphoreType.DMA,
    ),
)
def _sc_aggregate(xcat, srcs, dsts, zeros_h, agg_out,
                  src_v, dst_v, rows_v, agg_sh, sem0, sem1):
    c = lax.axis_index("c")
    s = lax.axis_index("s")

    pltpu.sync_copy(zeros_h, rows_v.at[0])
    for j in range(RPT // K):
        pltpu.sync_copy(rows_v.at[0], agg_sh.at[pl.ds(s * RPT + j * K, K)])
    plsc.subcore_barrier()

    def blk_body(blk, carry):
        pltpu.sync_copy(srcs.at[c].at[s].at[blk], src_v)
        pltpu.sync_copy(dsts.at[s].at[blk], dst_v)
        pltpu.async_copy(xcat.at[src_v.at[0]], rows_v.at[0], sem0)
        pltpu.async_copy(xcat.at[src_v.at[1]], rows_v.at[1], sem1)

        def pair_body(p, carry2):
            i = p * 2
            pltpu.make_async_copy(xcat.at[src_v.at[i]], rows_v.at[0], sem0).wait()
            pltpu.sync_copy(rows_v.at[0], agg_sh.at[dst_v.at[i]], add=True)
            pltpu.async_copy(xcat.at[src_v.at[i + 2]], rows_v.at[0], sem0)
            pltpu.make_async_copy(xcat.at[src_v.at[i + 1]], rows_v.at[1], sem1).wait()
            pltpu.sync_copy(rows_v.at[1], agg_sh.at[dst_v.at[i + 1]], add=True)
            pltpu.async_copy(xcat.at[src_v.at[i + 3]], rows_v.at[1], sem1)
            return carry2

        lax.fori_loop(0, CPB // 2 - 1, pair_body, ())
        i = CPB - 2
        pltpu.make_async_copy(xcat.at[src_v.at[i]], rows_v.at[0], sem0).wait()
        pltpu.sync_copy(rows_v.at[0], agg_sh.at[dst_v.at[i]], add=True)
        pltpu.make_async_copy(xcat.at[src_v.at[i + 1]], rows_v.at[1], sem1).wait()
        pltpu.sync_copy(rows_v.at[1], agg_sh.at[dst_v.at[i + 1]], add=True)
        return carry

    lax.fori_loop(0, NB, blk_body, ())
    plsc.subcore_barrier()

    for j in range(RPT // K):
        base = s * RPT + j * K
        pltpu.sync_copy(agg_sh.at[pl.ds(base, K)], rows_v.at[0])
        pltpu.sync_copy(rows_v.at[0], agg_out.at[c].at[pl.ds(base, K)])


def _tail_body(aggs, mskr, sb1, sb2, Wn, Wa_, va_, Wfc, bfc, Wd,
               preds_o, w_o, reg_o):
    ag = aggs[...]
    deg = ag[0, :N, DEG_COL:DEG_COL + 1]
    inv = 1.0 / jnp.maximum(deg, 1.0)
    agg1 = ag[0, :N, :D] * inv
    agg2 = ag[1, :N, :D] * inv
    Wnv = Wn[...]
    dot = functools.partial(jnp.dot, preferred_element_type=jnp.float32)
    h10 = jnp.maximum(dot(agg1, Wnv[0]), 0.0)
    h11 = jnp.maximum(dot(agg1, Wnv[1]), 0.0)
    h20 = jnp.maximum(dot(agg2, Wnv[0]), 0.0)
    h21 = jnp.maximum(dot(agg2, Wnv[1]), 0.0)

    Wav = Wa_[...]
    vav = va_[...]
    s0 = dot(jnp.tanh(dot(h10, Wav)), vav)
    s1 = dot(jnp.tanh(dot(h11, Wav)), vav)
    mx = jnp.maximum(s0, s1)
    e0 = jnp.exp(s0 - mx)
    e1 = jnp.exp(s1 - mx)
    z = e0 + e1
    a0 = e0 / z
    a1 = e1 / z
    outp = a0 * h10 + a1 * h11
    preds_o[...] = dot(outp, Wfc[...]) + bfc[...]
    w_o[...] = jnp.concatenate(
        [(jnp.sum(a0) / N).reshape(1, 1), (jnp.sum(a1) / N).reshape(1, 1)],
        axis=1)

    m = mskr[...]
    sm = jnp.sum(m)
    c0 = jax.nn.sigmoid(jnp.sum(h10 * m, axis=0, keepdims=True) / sm)
    c1 = jax.nn.sigmoid(jnp.sum(h11 * m, axis=0, keepdims=True) / sm)
    Wdv = Wd[...]
    cdims = (((1,), (1,)), ((), ()))
    u0 = lax.dot_general(c0, Wdv, cdims,
                         preferred_element_type=jnp.float32)
    u1 = lax.dot_general(c1, Wdv, cdims,
                         preferred_element_type=jnp.float32)
    s10 = lax.dot_general(u0, h10, cdims,
                          preferred_element_type=jnp.float32) + sb1[...]
    s11 = lax.dot_general(u1, h11, cdims,
                          preferred_element_type=jnp.float32) + sb1[...]
    s20 = lax.dot_general(u0, h20, cdims,
                          preferred_element_type=jnp.float32) + sb2[...]
    s21 = lax.dot_general(u1, h21, cdims,
                          preferred_element_type=jnp.float32) + sb2[...]
    reg_o[...] = jnp.concatenate(
        [jnp.concatenate([s10, s11], axis=0),
         jnp.concatenate([s20, s21], axis=0)], axis=1)


_tail = pl.pallas_call(
    _tail_body,
    out_shape=(
        jax.ShapeDtypeStruct((N, N_CLASSES), jnp.float32),
        jax.ShapeDtypeStruct((1, N_MP), jnp.float32),
        jax.ShapeDtypeStruct((N_MP, 2 * N), jnp.float32),
    ),
)


def kernel(feat1, feat2, msk, samp_bias1, samp_bias2, edge_index,
           W_prep, W_node, Wa, va, W_fc, b_fc, W_d):
    x2 = _prep(feat1, feat2, W_prep)
    xcat = x2.reshape(2 * N, DW)
    src = edge_index[0]
    dst = edge_index[1]
    srcs = jnp.stack([src, src + N]).reshape(NC, NT, NB, CPB, K)
    dsts = dst.reshape(NT, NB, CPB, K)
    zeros_h = jnp.zeros((K, DW), jnp.float32)
    aggs = _sc_aggregate(xcat, srcs, dsts, zeros_h)
    preds, w_o, reg = _tail(
        aggs, msk.reshape(N, 1),
        samp_bias1.reshape(1, N), samp_bias2.reshape(1, N),
        W_node, Wa, va.reshape(ATTN_DIM, 1), W_fc,
        b_fc.reshape(1, N_CLASSES), W_d)
    return preds, w_o.reshape(N_MP), reg


# speedup vs baseline: 11.6521x; 1.1528x over previous
import functools

import jax
import jax.numpy as jnp
from jax import lax
from jax.experimental import pallas as pl
from jax.experimental.pallas import tpu as pltpu
from jax.experimental.pallas import tpu_sc as plsc

N = 10000
E = 320000
D = 128
N_MP = 2
N_CLASSES = 16
ATTN_DIM = 64

NC = 2
NT = 16
EPT = E // NT
K = 80
NCH = EPT // K
NB = 5
CPB = NCH // NB
NP = 10240
RPT = NP // NT
DEGW = 16


def _prep_body(f1, f2, w, out):
    wv = w[...]
    out[0] = jnp.maximum(jnp.dot(f1[...], wv, preferred_element_type=jnp.float32), 0.0)
    out[1] = jnp.maximum(jnp.dot(f2[...], wv, preferred_element_type=jnp.float32), 0.0)


_prep = pl.pallas_call(
    _prep_body,
    out_shape=jax.ShapeDtypeStruct((2, N, D), jnp.float32),
)


_mesh = plsc.VectorSubcoreMesh(core_axis_name="c", subcore_axis_name="s")


@functools.partial(
    pl.kernel,
    out_type=(
        jax.ShapeDtypeStruct((NC, NP, D), jnp.float32),
        jax.ShapeDtypeStruct((NC, NP, DEGW), jnp.float32),
    ),
    mesh=_mesh,
    compiler_params=pltpu.CompilerParams(use_tc_tiling_on_sc=False),
    scratch_types=(
        pltpu.VMEM((CPB, K), jnp.int32),
        pltpu.VMEM((CPB, K), jnp.int32),
        pltpu.VMEM((2, K, D), jnp.float32),
        pltpu.VMEM((K, DEGW), jnp.float32),
        pltpu.VMEM((K, DEGW), jnp.float32),
        pltpu.VMEM_SHARED((NP, D), jnp.float32),
        pltpu.VMEM_SHARED((NP, DEGW), jnp.float32),
        pltpu.SemaphoreType.DMA,
        pltpu.SemaphoreType.DMA,
    ),
)
def _sc_aggregate(xcat, srcs, dsts, ones_h, zeros_h, zdeg_h, agg_out, deg_out,
                  src_v, dst_v, rows_v, ones_v, zdeg_v, agg_sh, deg_sh,
                  sem0, sem1):
    c = lax.axis_index("c")
    s = lax.axis_index("s")
    xc = xcat.at[c]

    pltpu.sync_copy(ones_h, ones_v)
    pltpu.sync_copy(zdeg_h, zdeg_v)
    pltpu.sync_copy(zeros_h, rows_v.at[0])
    for j in range(RPT // K):
        pltpu.sync_copy(rows_v.at[0], agg_sh.at[pl.ds(s * RPT + j * K, K)])
        pltpu.sync_copy(zdeg_v, deg_sh.at[pl.ds(s * RPT + j * K, K)])
    plsc.subcore_barrier()

    def blk_body(blk, carry):
        pltpu.sync_copy(srcs.at[s].at[blk], src_v)
        pltpu.sync_copy(dsts.at[s].at[blk], dst_v)
        pltpu.async_copy(xc.at[src_v.at[0]], rows_v.at[0], sem0)
        pltpu.async_copy(xc.at[src_v.at[1]], rows_v.at[1], sem1)

        def even_chunk(i):
            pltpu.make_async_copy(xc.at[src_v.at[i]], rows_v.at[0], sem0).wait()
            pltpu.sync_copy(rows_v.at[0], agg_sh.at[dst_v.at[i]], add=True)

            @pl.when(c == 0)
            def _():
                pltpu.sync_copy(ones_v, deg_sh.at[dst_v.at[i]], add=True)

        def odd_chunk(i):
            pltpu.make_async_copy(xc.at[src_v.at[i]], rows_v.at[1], sem1).wait()
            pltpu.sync_copy(rows_v.at[1], agg_sh.at[dst_v.at[i]], add=True)

            @pl.when(c == 1)
            def _():
                pltpu.sync_copy(ones_v, deg_sh.at[dst_v.at[i]], add=True)

        def pair_body(p, carry2):
            i = p * 2
            even_chunk(i)
            pltpu.async_copy(xc.at[src_v.at[i + 2]], rows_v.at[0], sem0)
            odd_chunk(i + 1)
            pltpu.async_copy(xc.at[src_v.at[i + 3]], rows_v.at[1], sem1)
            return carry2

        lax.fori_loop(0, CPB // 2 - 1, pair_body, ())
        even_chunk(CPB - 2)
        odd_chunk(CPB - 1)
        return carry

    lax.fori_loop(0, NB, blk_body, ())
    plsc.subcore_barrier()

    for j in range(RPT // K):
        base = s * RPT + j * K
        pltpu.sync_copy(agg_sh.at[pl.ds(base, K)], rows_v.at[0])
        pltpu.sync_copy(rows_v.at[0], agg_out.at[c].at[pl.ds(base, K)])
        pltpu.sync_copy(deg_sh.at[pl.ds(base, K)], zdeg_v)
        pltpu.sync_copy(zdeg_v, deg_out.at[c].at[pl.ds(base, K)])


def _tail_body(aggs, degp, mskr, sb1, sb2, Wn, Wa_, va_, Wfc, bfc, Wd,
               preds_o, w_o, reg_o):
    dp = degp[...]
    deg = dp[0, :N, 0:1] + dp[1, :N, 0:1]
    inv = 1.0 / jnp.maximum(deg, 1.0)
    ag = aggs[...]
    agg1 = ag[0, :N] * inv
    agg2 = ag[1, :N] * inv
    Wnv = Wn[...]
    dot = functools.partial(jnp.dot, preferred_element_type=jnp.float32)
    h10 = jnp.maximum(dot(agg1, Wnv[0]), 0.0)
    h11 = jnp.maximum(dot(agg1, Wnv[1]), 0.0)
    h20 = jnp.maximum(dot(agg2, Wnv[0]), 0.0)
    h21 = jnp.maximum(dot(agg2, Wnv[1]), 0.0)

    Wav = Wa_[...]
    vav = va_[...]
    s0 = dot(jnp.tanh(dot(h10, Wav)), vav)
    s1 = dot(jnp.tanh(dot(h11, Wav)), vav)
    mx = jnp.maximum(s0, s1)
    e0 = jnp.exp(s0 - mx)
    e1 = jnp.exp(s1 - mx)
    z = e0 + e1
    a0 = e0 / z
    a1 = e1 / z
    outp = a0 * h10 + a1 * h11
    preds_o[...] = dot(outp, Wfc[...]) + bfc[...]
    w_o[...] = jnp.concatenate(
        [(jnp.sum(a0) / N).reshape(1, 1), (jnp.sum(a1) / N).reshape(1, 1)],
        axis=1)

    m = mskr[...]
    sm = jnp.sum(m)
    c0 = jax.nn.sigmoid(jnp.sum(h10 * m, axis=0, keepdims=True) / sm)
    c1 = jax.nn.sigmoid(jnp.sum(h11 * m, axis=0, keepdims=True) / sm)
    Wdv = Wd[...]
    cdims = (((1,), (1,)), ((), ()))
    u0 = lax.dot_general(c0, Wdv, cdims,
                         preferred_element_type=jnp.float32)
    u1 = lax.dot_general(c1, Wdv, cdims,
                         preferred_element_type=jnp.float32)
    s10 = lax.dot_general(u0, h10, cdims,
                          preferred_element_type=jnp.float32) + sb1[...]
    s11 = lax.dot_general(u1, h11, cdims,
                          preferred_element_type=jnp.float32) + sb1[...]
    s20 = lax.dot_general(u0, h20, cdims,
                          preferred_element_type=jnp.float32) + sb2[...]
    s21 = lax.dot_general(u1, h21, cdims,
                          preferred_element_type=jnp.float32) + sb2[...]
    reg_o[...] = jnp.concatenate(
        [jnp.concatenate([s10, s11], axis=0),
         jnp.concatenate([s20, s21], axis=0)], axis=1)


_tail = pl.pallas_call(
    _tail_body,
    out_shape=(
        jax.ShapeDtypeStruct((N, N_CLASSES), jnp.float32),
        jax.ShapeDtypeStruct((1, N_MP), jnp.float32),
        jax.ShapeDtypeStruct((N_MP, 2 * N), jnp.float32),
    ),
)


def kernel(feat1, feat2, msk, samp_bias1, samp_bias2, edge_index,
           W_prep, W_node, Wa, va, W_fc, b_fc, W_d):
    xcat = _prep(feat1, feat2, W_prep)
    srcs = edge_index[0].reshape(NT, NB, CPB, K)
    dsts = edge_index[1].reshape(NT, NB, CPB, K)
    ones_h = jnp.ones((K, DEGW), jnp.float32)
    zeros_h = jnp.zeros((K, D), jnp.float32)
    zdeg_h = jnp.zeros((K, DEGW), jnp.float32)
    aggs, degp = _sc_aggregate(xcat, srcs, dsts, ones_h, zeros_h, zdeg_h)
    preds, w_o, reg = _tail(
        aggs, degp, msk.reshape(N, 1),
        samp_bias1.reshape(1, N), samp_bias2.reshape(1, N),
        W_node, Wa, va.reshape(ATTN_DIM, 1), W_fc,
        b_fc.reshape(1, N_CLASSES), W_d)
    return preds, w_o.reshape(N_MP), reg
